# baseline (speedup 1.0000x reference)
import jax
import jax.numpy as jnp
from jax import lax
from jax.experimental import pallas as pl
from jax.experimental.pallas import tpu as pltpu
from jax.experimental.pallas import tpu_sc as plsc

N = 10000
E = 320000
D_IN = 128
D_H = 256
D_OUT = 64

T = 16
EPT = E // T
CHUNK = 2000
NCHUNK = EPT // CHUNK
VPC = CHUNK // 16
SCAP = N + 16
NVEC = N // 16
MCAP = CHUNK + 16
NPAD = 10240
SMAX = 1024

_mesh = plsc.VectorSubcoreMesh(
    core_axis_name="c", subcore_axis_name="s", num_cores=1, num_subcores=T
)


def _rsqrt(x):
  i = plsc.bitcast(x, jnp.int32)
  y = plsc.bitcast(jnp.int32(0x5F3759DF) - (i >> 1), jnp.float32)
  for _ in range(4):
    y = y * (1.5 - 0.5 * x * y * y)
  return y


def _sc_body(
    ei_hbm, x_hbm, w1_hbm, b1_hbm,
    w2_hbm, b2_hbm, wh1_hbm, bh1_hbm, wh2_hbm, bh2_hbm,
    z_hbm, l0_hbm, hist_hbm,
    dbuf, sbuf, dbuf2, sbuf2, sem0, sem1, dis_v, flag_v, l0buf, slotnode_v, w_v,
    msrc, mslot, mnrm, idxg, slotg, rows_v,
    w1_v, b1_v, zblk, zfull, w2_v, wh1_v, wh2_v, b2blk, bh1blk, bh2blk,
    vec16, cntall_v, degbuf, hbuf, hsem,
    dis_sh, flag_sh, slotnode_sh, w_sh, meta_sh, cnt_sh, agg_sh, z_sh,
):
  t = lax.axis_index("s")
  iota = lax.iota(jnp.int32, 16)
  fzero16 = jnp.zeros((16,), jnp.float32)
  izero16 = jnp.zeros((16,), jnp.int32)

  def _z(i, c):
    dis_v[pl.ds(i * 16, 16)] = fzero16
    flag_v[pl.ds(i * 16, 16)] = izero16
    return c
  lax.fori_loop(0, NVEC, _z, 0)

  def _z2(i, c):
    w_v[pl.ds(i * 16, 16)] = fzero16
    slotnode_v[pl.ds(i * 16, 16)] = izero16
    return c
  lax.fori_loop(0, SCAP // 16, _z2, 0)

  for l in range(16):
    def _zr(b, c, l=l):
      rows_v[l, pl.ds(b * 16, 16)] = fzero16
      return c
    lax.fori_loop(0, 8, _zr, 0)
  zblk[...] = fzero16

  def _edma(cidx, db, sb, sem):
    base = pl.multiple_of((t * NCHUNK + cidx) * CHUNK, 8)
    d1 = pltpu.make_async_copy(
        ei_hbm.at[pl.ds(E + base, CHUNK)], db.at[pl.ds(0, CHUNK)], sem)
    d2 = pltpu.make_async_copy(ei_hbm.at[pl.ds(base, CHUNK)], sb, sem)
    return d1, d2

  def _fire(cidx, db, sb, sem):
    d1, d2 = _edma(cidx, db, sb, sem)
    d1.start()
    d2.start()

  def _drain(cidx, db, sb, sem):
    d1, d2 = _edma(cidx, db, sb, sem)
    d1.wait()
    d2.wait()

  def _scan_pipe(chunk_body, init):
    _fire(0, dbuf, sbuf, sem0)
    _fire(1, dbuf2, sbuf2, sem1)

    def _pair(pp, carry):
      c0 = pp * 2
      _drain(c0, dbuf, sbuf, sem0)
      carry = chunk_body(dbuf, sbuf, carry)
      _fire(c0 + 2, dbuf, sbuf, sem0)
      _drain(c0 + 1, dbuf2, sbuf2, sem1)
      carry = chunk_body(dbuf2, sbuf2, carry)
      _fire(c0 + 3, dbuf2, sbuf2, sem1)
      return carry
    carry = lax.fori_loop(0, NCHUNK // 2 - 1, _pair, init)
    _drain(NCHUNK - 2, dbuf, sbuf, sem0)
    carry = chunk_body(dbuf, sbuf, carry)
    _drain(NCHUNK - 1, dbuf2, sbuf2, sem1)
    carry = chunk_body(dbuf2, sbuf2, carry)
    return carry


  def _chunk_a(db, sb, cnt0v):
    def _vec(i, cv):
      d = db[pl.ds(i * 16, 16)]
      cntv, lastm = plsc.scan_count(d)
      plsc.addupdate_scatter(
          dis_v, [d], cntv.astype(jnp.float32), mask=lastm)
      m = d == 0
      s = sb[pl.ds(i * 16, 16)]
      pc = plsc.cumsum(m.astype(jnp.int32))
      pos = pc - 1 + cv
      plsc.store_scatter(l0buf, [pos], s, mask=m)
      return cv + plsc.all_reduce_population_count(m)

    return lax.fori_loop(0, VPC, _vec, cnt0v, unroll=4)

  cnt0v = _scan_pipe(_chunk_a, izero16)
  cnt0 = cnt0v[0]

  pltpu.sync_copy(dis_v.at[pl.ds(0, N)],
                  hist_hbm.at[pl.ds(pl.multiple_of(t * NPAD, 8), N)])
  pltpu.sync_copy(l0buf, l0_hbm.at[pl.ds(pl.multiple_of(t * EPT, 8), EPT)])
  vec16[...] = jnp.full((16,), cnt0, jnp.int32)
  pltpu.sync_copy(vec16, cnt_sh.at[pl.ds(pl.multiple_of(t * 16, 8), 16)])
  plsc.subcore_barrier()

  copies = [
      pltpu.make_async_copy(
          hist_hbm.at[pl.ds(pl.multiple_of(tt * NPAD + t * 640, 8), 640)],
          hbuf.at[pl.ds(tt * 640, 640)], hsem)
      for tt in range(T)
  ]
  for cp in copies:
    cp.start()
  for cp in copies:
    cp.wait()

  def _acc(i, c2):
    acc = hbuf[pl.ds(i * 16, 16)]
    for tt in range(1, T):
      acc = acc + hbuf[pl.ds(tt * 640 + i * 16, 16)]
    degbuf[pl.ds(i * 16, 16)] = acc
    return c2
  lax.fori_loop(0, 40, _acc, 0)

  def _dis(i, c):
    dv = degbuf[pl.ds(i * 16, 16)] + 1.0
    degbuf[pl.ds(i * 16, 16)] = _rsqrt(dv)
    return c
  lax.fori_loop(0, 40, _dis, 0)
  pltpu.sync_copy(degbuf, dis_sh.at[pl.ds(pl.multiple_of(t * 640, 8), 640)])
  plsc.subcore_barrier()
  pltpu.sync_copy(dis_sh.at[pl.ds(0, N)], dis_v.at[pl.ds(0, N)])

  lane0 = iota == 0

  def _sstore(ref, idx, val):
    plsc.store_scatter(
        ref, [jnp.full((16,), idx, jnp.int32)],
        jnp.full((16,), val, ref.dtype), mask=lane0)

  @pl.when(t == 0)
  def _dedup():
    pltpu.sync_copy(cnt_sh, cntall_v)
    _sstore(flag_v, jnp.int32(0), jnp.int32(1))

    def _tile(tt, ns):
      cnt_t = cntall_v[pl.ds(tt * 16, 16)][0]

      def _chunk(c, ns):
        cbase = pl.multiple_of((tt * NCHUNK + c) * CHUNK, 8)
        pltpu.sync_copy(l0_hbm.at[pl.ds(cbase, CHUNK)],
                        dbuf.at[pl.ds(0, CHUNK)])
        kmax = jnp.minimum(jnp.int32(CHUNK), cnt_t - c * CHUNK)

        def _k(k, ns):
          s = dbuf[pl.ds(k, 16)][0]
          f = flag_v[pl.ds(s, 16)][0]
          isnew = (f == 0).astype(jnp.int32)
          slot = jnp.where(f == 0, ns, f - 1)
          _sstore(flag_v, s, slot + 1)
          _sstore(slotnode_v, slot, s)
          wnew = w_v[pl.ds(slot, 16)][0] + dis_v[pl.ds(s, 16)][0]
          _sstore(w_v, slot, wnew)
          return ns + isnew

        return lax.fori_loop(0, kmax, _k, ns)

      nchunks = (cnt_t + CHUNK - 1) // CHUNK
      return lax.fori_loop(0, nchunks, _chunk, ns)

    ns = lax.fori_loop(0, T, _tile, jnp.int32(1))
    pltpu.sync_copy(flag_v.at[pl.ds(0, N)], flag_sh)
    pltpu.sync_copy(slotnode_v, slotnode_sh)
    pltpu.sync_copy(w_v, w_sh)
    vec16[...] = jnp.full((16,), ns, jnp.int32)
    pltpu.sync_copy(vec16, meta_sh)

  plsc.subcore_barrier()

  pltpu.sync_copy(flag_sh, flag_v.at[pl.ds(0, N)])
  pltpu.sync_copy(slotnode_sh, slotnode_v)
  pltpu.sync_copy(w_sh, w_v)
  pltpu.sync_copy(meta_sh, vec16)
  nslots = vec16[...][0]
  dis0 = dis_v[pl.ds(0, 16)][0]
  pltpu.sync_copy(w1_hbm.at[pl.ds(pl.multiple_of(t * (D_IN * 16), 8),
                                  D_IN * 16)], w1_v)
  pltpu.sync_copy(b1_hbm.at[pl.ds(pl.multiple_of(t * 16, 8), 16)], b1_v)

  def _process16(srcv, slotv, nrmv):
    idxg[...] = srcv
    slotg[...] = slotv
    pltpu.sync_copy(x_hbm.at[idxg], rows_v)

    for l in range(16):
      nl = nrmv[l]

      def _b(b, c2, l=l, nl=nl):
        v = rows_v[l, pl.ds(b * 16, 16)]
        rows_v[l, pl.ds(b * 16, 16)] = v * nl
        return c2
      lax.fori_loop(0, 8, _b, 0)
    pltpu.sync_copy(rows_v, agg_sh.at[slotg], add=True)

  ngroups = (nslots + SMAX - 1) // SMAX

  def _group(g, c):
    glo = g * SMAX
    gcount = jnp.minimum(nslots - glo, jnp.int32(SMAX))

    for l in range(16):
      def _zr2(b, c2, l=l):
        rows_v[l, pl.ds(b * 16, 16)] = fzero16
        return c2
      lax.fori_loop(0, 8, _zr2, 0)

    mv = (gcount + 15) // 16

    def _za(k, c2):
      mchunk = k * 16 + t
      rvec = mchunk * 16 + iota
      rz = jnp.where(rvec < gcount, rvec, jnp.int32(SMAX))
      slotg[...] = rz
      pltpu.sync_copy(rows_v, agg_sh.at[slotg])
      return c2
    lax.fori_loop(0, jnp.maximum(0, (mv - t + 15) // 16), _za, 0)
    plsc.subcore_barrier()

    def _chunk_e(db, sb, cc):
      def _vec(i, mcv):
        d = db[pl.ds(i * 16, 16)]
        f = plsc.load_gather(flag_v, [d])
        gs = f - 1 - glo
        m = (f > 0) & (gs >= 0) & (gs < gcount)
        s = sb[pl.ds(i * 16, 16)]
        nrm = plsc.load_gather(dis_v, [s]) * plsc.load_gather(dis_v, [d])
        pc = plsc.cumsum(m.astype(jnp.int32))
        pos = pc - 1 + mcv
        plsc.store_scatter(msrc, [pos], s, mask=m)
        plsc.store_scatter(mslot, [pos], gs, mask=m)
        plsc.store_scatter(mnrm, [pos], nrm, mask=m)
        return mcv + plsc.all_reduce_population_count(m)

      mcv = lax.fori_loop(0, VPC, _vec, izero16, unroll=4)
      mcnt = mcv[0]

      flo = (mcnt // 16) * 16
      padm = (iota + flo) >= mcnt
      plsc.store_scatter(msrc, [iota + flo], izero16, mask=padm)
      plsc.store_scatter(mslot, [iota + flo],
                         jnp.full((16,), SMAX, jnp.int32), mask=padm)
      plsc.store_scatter(mnrm, [iota + flo], fzero16, mask=padm)

      def _bat(r, c2):
        _process16(
            msrc[pl.ds(r * 16, 16)],
            mslot[pl.ds(r * 16, 16)],
            mnrm[pl.ds(r * 16, 16)],
        )
        return c2
      lax.fori_loop(0, (mcnt + 15) // 16, _bat, 0)
      return cc

    _scan_pipe(_chunk_e, 0)

    gv = (gcount + 15) // 16

    def _selfk(k, c2):
      v = k * 16 + t
      gslot = v * 16 + iota
      jvec = glo + gslot
      m = gslot < gcount
      nodes = plsc.load_gather(slotnode_v, [jvec], mask=m)
      nodes = jnp.where(m, nodes, 0)
      dv = plsc.load_gather(dis_v, [nodes])
      nrm = jnp.where(m, dv * dv, fzero16)
      slots = jnp.where(m, gslot, jnp.int32(SMAX))
      _process16(nodes, slots, nrm)
      return c2
    lax.fori_loop(0, jnp.maximum(0, (gv - t + 15) // 16), _selfk, 0)
    plsc.subcore_barrier()

    def _fb(r0, c2):
      rvec = r0 * 16 + iota
      rz = jnp.where(rvec < gcount, rvec, 0)
      idxg[...] = rz
      pltpu.sync_copy(agg_sh.at[idxg], rows_v)
      zreg = zblk[...]
      for l in range(16):
        acc = b1_v[...]

        def _kv(kv, acc, l=l):
          av = rows_v[l, pl.ds(kv * 16, 16)]
          for lane in range(16):
            acc = acc + av[lane] * w1_v[pl.ds((kv * 16 + lane) * 16, 16)]
          return acc
        acc = lax.fori_loop(0, D_IN // 16, _kv, acc)
        h = jnp.maximum(acc, 0.0)
        j = glo + r0 * 16 + l
        valid = (r0 * 16 + l < gcount).astype(jnp.float32)
        wj = w_v[pl.ds(j, 16)][0]
        wt = (dis0 * wj
              + jnp.where(j == 0, dis0 * dis0, jnp.float32(0.0))) * valid
        zreg = zreg + wt * h
      zblk[...] = zreg
      return c2
    lax.fori_loop(0, (gcount + 15) // 16, _fb, 0)
    plsc.subcore_barrier()
    return c

  lax.fori_loop(0, ngroups, _group, 0)

  pltpu.sync_copy(zblk, z_sh.at[pl.ds(pl.multiple_of(t * 16, 8), 16)])
  pltpu.sync_copy(w2_hbm.at[pl.ds(pl.multiple_of(t * (D_H * 16), 8),
                                  D_H * 16)], w2_v)
  pltpu.sync_copy(b2_hbm.at[pl.ds(pl.multiple_of(t * 16, 8), 16)], b2blk)
  pltpu.sync_copy(wh1_hbm.at[pl.ds(pl.multiple_of(t * (D_H * 16), 8),
                                   D_H * 16)], wh1_v)
  pltpu.sync_copy(bh1_hbm.at[pl.ds(pl.multiple_of(t * 16, 8), 16)], bh1blk)
  tq = jnp.minimum(t, 3)
  pltpu.sync_copy(wh2_hbm.at[pl.ds(pl.multiple_of(tq * (D_H * 16), 8),
                                   D_H * 16)], wh2_v)
  pltpu.sync_copy(bh2_hbm.at[pl.ds(pl.multiple_of(tq * 16, 8), 16)], bh2blk)
  plsc.subcore_barrier()

  def _mv256(wref, bias):
    def _kv(kv, acc):
      av = zfull[pl.ds(kv * 16, 16)]
      for lane in range(16):
        acc = acc + av[lane] * wref[pl.ds((kv * 16 + lane) * 16, 16)]
      return acc
    return lax.fori_loop(0, D_H // 16, _kv, bias)

  pltpu.sync_copy(z_sh, zfull)
  zblk[...] = jnp.maximum(_mv256(w2_v, b2blk[...]), 0.0)
  pltpu.sync_copy(zblk, z_sh.at[pl.ds(pl.multiple_of(t * 16, 8), 16)])
  plsc.subcore_barrier()

  pltpu.sync_copy(z_sh, zfull)
  zblk[...] = jnp.maximum(_mv256(wh1_v, bh1blk[...]), 0.0)
  pltpu.sync_copy(zblk, z_sh.at[pl.ds(pl.multiple_of(t * 16, 8), 16)])
  plsc.subcore_barrier()

  pltpu.sync_copy(z_sh, zfull)

  @pl.when(t < 4)
  def _qblk():
    zblk[...] = _mv256(wh2_v, bh2blk[...])
    pltpu.sync_copy(zblk, z_sh.at[pl.ds(pl.multiple_of(t * 16, 8), 16)])
  plsc.subcore_barrier()

  @pl.when(t == 0)
  def _finish():
    pltpu.sync_copy(z_sh.at[pl.ds(0, D_OUT)], zfull.at[pl.ds(0, D_OUT)])
    pltpu.sync_copy(zfull.at[pl.ds(0, D_OUT)], z_hbm)


_sc_kernel = pl.kernel(
    _sc_body,
    out_type=(
        jax.ShapeDtypeStruct((D_OUT,), jnp.float32),
        jax.ShapeDtypeStruct((E,), jnp.int32),
        jax.ShapeDtypeStruct((T * NPAD,), jnp.float32),
    ),
    mesh=_mesh,
    compiler_params=pltpu.CompilerParams(needs_layout_passes=False),
    scratch_types=[
        pltpu.VMEM((MCAP,), jnp.int32),
        pltpu.VMEM((CHUNK,), jnp.int32),
        pltpu.VMEM((MCAP,), jnp.int32),
        pltpu.VMEM((CHUNK,), jnp.int32),
        pltpu.SemaphoreType.DMA,
        pltpu.SemaphoreType.DMA,
        pltpu.VMEM((N + 16,), jnp.float32),
        pltpu.VMEM((N + 16,), jnp.int32),
        pltpu.VMEM((EPT,), jnp.int32),
        pltpu.VMEM((SCAP,), jnp.int32),
        pltpu.VMEM((SCAP,), jnp.float32),
        pltpu.VMEM((MCAP,), jnp.int32),
        pltpu.VMEM((MCAP,), jnp.int32),
        pltpu.VMEM((MCAP,), jnp.float32),
        pltpu.VMEM((16,), jnp.int32),
        pltpu.VMEM((16,), jnp.int32),
        pltpu.VMEM((16, D_IN), jnp.float32),
        pltpu.VMEM((D_IN * 16,), jnp.float32),
        pltpu.VMEM((16,), jnp.float32),
        pltpu.VMEM((16,), jnp.float32),
        pltpu.VMEM((D_H,), jnp.float32),
        pltpu.VMEM((D_IN * 32,), jnp.float32),
        pltpu.VMEM((D_IN * 32,), jnp.float32),
        pltpu.VMEM((D_IN * 32,), jnp.float32),
        pltpu.VMEM((16,), jnp.float32),
        pltpu.VMEM((16,), jnp.float32),
        pltpu.VMEM((16,), jnp.float32),
        pltpu.VMEM((16,), jnp.int32),
        pltpu.VMEM((T * 16,), jnp.int32),
        pltpu.VMEM((640,), jnp.float32),
        pltpu.VMEM((T * 640,), jnp.float32),
        pltpu.SemaphoreType.DMA,
        pltpu.VMEM_SHARED((NPAD,), jnp.float32),
        pltpu.VMEM_SHARED((N,), jnp.int32),
        pltpu.VMEM_SHARED((SCAP,), jnp.int32),
        pltpu.VMEM_SHARED((SCAP,), jnp.float32),
        pltpu.VMEM_SHARED((16,), jnp.int32),
        pltpu.VMEM_SHARED((T * 16,), jnp.int32),
        pltpu.VMEM_SHARED((SMAX + 8, D_IN), jnp.float32),
        pltpu.VMEM_SHARED((D_H,), jnp.float32),
    ],
)


def _blocks(w):
  k, c = w.shape
  return w.reshape(k, c // 16, 16).transpose(1, 0, 2).reshape(-1)


def kernel(x, edge_index, W1, b1, W2, b2, Wh1, bh1, Wh2, bh2):
  q, _, _ = _sc_kernel(
      edge_index.reshape(-1), x, _blocks(W1), b1,
      _blocks(W2), b2, _blocks(Wh1), bh1, _blocks(Wh2), bh2)
  return q


# speedup vs baseline: 1.1412x; 1.1412x over previous
import jax
import jax.numpy as jnp
from jax import lax
from jax.experimental import pallas as pl
from jax.experimental.pallas import tpu as pltpu
from jax.experimental.pallas import tpu_sc as plsc

N = 10000
E = 320000
D_IN = 128
D_H = 256
D_OUT = 64

T = 16
EPT = E // T
CHUNK = 2000
NCHUNK = EPT // CHUNK
VPC = CHUNK // 16
SCAP = N + 16
NVEC = N // 16
MCAP = CHUNK + 16
NPAD = 10240
SMAX = 1024

_mesh = plsc.VectorSubcoreMesh(
    core_axis_name="c", subcore_axis_name="s", num_cores=1, num_subcores=T
)


def _rsqrt(x):
  i = plsc.bitcast(x, jnp.int32)
  y = plsc.bitcast(jnp.int32(0x5F3759DF) - (i >> 1), jnp.float32)
  for _ in range(4):
    y = y * (1.5 - 0.5 * x * y * y)
  return y


def _sc_body(
    ei_hbm, x_hbm, w1_hbm, b1_hbm,
    z_hbm, l0_hbm, hist_hbm,
    dbuf, sbuf, dbuf2, sbuf2, sem0, sem1, dis_v, flag_v, l0buf, slotnode_v, w_v,
    msrc, mslot, mnrm, idxg, slotg, rows_v,
    w1_v, b1_v, zblk, zfull, vec16, cntall_v, degbuf, hbuf, hsem,
    dis_sh, flag_sh, slotnode_sh, w_sh, meta_sh, cnt_sh, agg_sh, z_sh,
):
  t = lax.axis_index("s")
  iota = lax.iota(jnp.int32, 16)
  fzero16 = jnp.zeros((16,), jnp.float32)
  izero16 = jnp.zeros((16,), jnp.int32)

  def _z(i, c):
    dis_v[pl.ds(i * 16, 16)] = fzero16
    flag_v[pl.ds(i * 16, 16)] = izero16
    return c
  lax.fori_loop(0, NVEC, _z, 0)

  def _z2(i, c):
    w_v[pl.ds(i * 16, 16)] = fzero16
    slotnode_v[pl.ds(i * 16, 16)] = izero16
    return c
  lax.fori_loop(0, SCAP // 16, _z2, 0)

  for l in range(16):
    def _zr(b, c, l=l):
      rows_v[l, pl.ds(b * 16, 16)] = fzero16
      return c
    lax.fori_loop(0, 8, _zr, 0)
  zblk[...] = fzero16

  def _edma(cidx, db, sb, sem):
    base = pl.multiple_of((t * NCHUNK + cidx) * CHUNK, 8)
    d1 = pltpu.make_async_copy(
        ei_hbm.at[pl.ds(E + base, CHUNK)], db.at[pl.ds(0, CHUNK)], sem)
    d2 = pltpu.make_async_copy(ei_hbm.at[pl.ds(base, CHUNK)], sb, sem)
    return d1, d2

  def _fire(cidx, db, sb, sem):
    d1, d2 = _edma(cidx, db, sb, sem)
    d1.start()
    d2.start()

  def _drain(cidx, db, sb, sem):
    d1, d2 = _edma(cidx, db, sb, sem)
    d1.wait()
    d2.wait()

  def _scan_pipe(chunk_body, init):
    _fire(0, dbuf, sbuf, sem0)
    _fire(1, dbuf2, sbuf2, sem1)

    def _pair(pp, carry):
      c0 = pp * 2
      _drain(c0, dbuf, sbuf, sem0)
      carry = chunk_body(dbuf, sbuf, carry)
      _fire(c0 + 2, dbuf, sbuf, sem0)
      _drain(c0 + 1, dbuf2, sbuf2, sem1)
      carry = chunk_body(dbuf2, sbuf2, carry)
      _fire(c0 + 3, dbuf2, sbuf2, sem1)
      return carry
    carry = lax.fori_loop(0, NCHUNK // 2 - 1, _pair, init)
    _drain(NCHUNK - 2, dbuf, sbuf, sem0)
    carry = chunk_body(dbuf, sbuf, carry)
    _drain(NCHUNK - 1, dbuf2, sbuf2, sem1)
    carry = chunk_body(dbuf2, sbuf2, carry)
    return carry


  def _chunk_a(db, sb, cnt0v):
    def _avec(o, cv, d):
      m = d == 0
      s = sb[pl.ds(o, 16)]
      pc = plsc.cumsum(m.astype(jnp.int32))
      plsc.store_scatter(l0buf, [pc - 1 + cv], s, mask=m)
      return cv + plsc.all_reduce_population_count(m)

    def _pair(i, cv):
      o = i * 32
      d1 = db[pl.ds(o, 16)]
      d2 = db[pl.ds(o + 16, 16)]
      c1, l1 = plsc.scan_count(d1)
      c2, l2 = plsc.scan_count(d2)
      plsc.addupdate_scatter(dis_v, [d1], c1.astype(jnp.float32), mask=l1)
      plsc.addupdate_scatter(dis_v, [d2], c2.astype(jnp.float32), mask=l2)
      cv = _avec(o, cv, d1)
      return _avec(o + 16, cv, d2)
    cv = lax.fori_loop(0, VPC // 2, _pair, cnt0v, unroll=2)
    o = (VPC - 1) * 16
    d = db[pl.ds(o, 16)]
    c1, l1 = plsc.scan_count(d)
    plsc.addupdate_scatter(dis_v, [d], c1.astype(jnp.float32), mask=l1)
    return _avec(o, cv, d)

  cnt0v = _scan_pipe(_chunk_a, izero16)
  cnt0 = cnt0v[0]

  pltpu.sync_copy(dis_v.at[pl.ds(0, N)],
                  hist_hbm.at[pl.ds(pl.multiple_of(t * NPAD, 8), N)])
  pltpu.sync_copy(l0buf, l0_hbm.at[pl.ds(pl.multiple_of(t * EPT, 8), EPT)])
  vec16[...] = jnp.full((16,), cnt0, jnp.int32)
  pltpu.sync_copy(vec16, cnt_sh.at[pl.ds(pl.multiple_of(t * 16, 8), 16)])
  plsc.subcore_barrier()

  copies = [
      pltpu.make_async_copy(
          hist_hbm.at[pl.ds(pl.multiple_of(tt * NPAD + t * 640, 8), 640)],
          hbuf.at[pl.ds(tt * 640, 640)], hsem)
      for tt in range(T)
  ]
  for cp in copies:
    cp.start()
  for cp in copies:
    cp.wait()

  def _acc(i, c2):
    acc = hbuf[pl.ds(i * 16, 16)]
    for tt in range(1, T):
      acc = acc + hbuf[pl.ds(tt * 640 + i * 16, 16)]
    degbuf[pl.ds(i * 16, 16)] = acc
    return c2
  lax.fori_loop(0, 40, _acc, 0)

  def _dis(i, c):
    dv = degbuf[pl.ds(i * 16, 16)] + 1.0
    degbuf[pl.ds(i * 16, 16)] = _rsqrt(dv)
    return c
  lax.fori_loop(0, 40, _dis, 0)
  pltpu.sync_copy(degbuf, dis_sh.at[pl.ds(pl.multiple_of(t * 640, 8), 640)])
  plsc.subcore_barrier()
  pltpu.sync_copy(dis_sh.at[pl.ds(0, N)], dis_v.at[pl.ds(0, N)])

  lane0 = iota == 0

  def _sstore(ref, idx, val):
    plsc.store_scatter(
        ref, [jnp.full((16,), idx, jnp.int32)],
        jnp.full((16,), val, ref.dtype), mask=lane0)

  @pl.when(t == 0)
  def _dedup():
    pltpu.sync_copy(cnt_sh, cntall_v)
    _sstore(flag_v, jnp.int32(0), jnp.int32(1))

    def _tile(tt, ns):
      cnt_t = cntall_v[pl.ds(tt * 16, 16)][0]

      def _chunk(c, ns):
        cbase = pl.multiple_of((tt * NCHUNK + c) * CHUNK, 8)
        pltpu.sync_copy(l0_hbm.at[pl.ds(cbase, CHUNK)],
                        dbuf.at[pl.ds(0, CHUNK)])
        kmax = jnp.minimum(jnp.int32(CHUNK), cnt_t - c * CHUNK)

        def _k(k, ns):
          s = dbuf[pl.ds(k, 16)][0]
          f = flag_v[pl.ds(s, 16)][0]
          isnew = (f == 0).astype(jnp.int32)
          slot = jnp.where(f == 0, ns, f - 1)
          _sstore(flag_v, s, slot + 1)
          _sstore(slotnode_v, slot, s)
          wnew = w_v[pl.ds(slot, 16)][0] + dis_v[pl.ds(s, 16)][0]
          _sstore(w_v, slot, wnew)
          return ns + isnew

        return lax.fori_loop(0, kmax, _k, ns)

      nchunks = (cnt_t + CHUNK - 1) // CHUNK
      return lax.fori_loop(0, nchunks, _chunk, ns)

    ns = lax.fori_loop(0, T, _tile, jnp.int32(1))
    pltpu.sync_copy(flag_v.at[pl.ds(0, N)], flag_sh)
    pltpu.sync_copy(slotnode_v, slotnode_sh)
    pltpu.sync_copy(w_v, w_sh)
    vec16[...] = jnp.full((16,), ns, jnp.int32)
    pltpu.sync_copy(vec16, meta_sh)

  plsc.subcore_barrier()

  pltpu.sync_copy(flag_sh, flag_v.at[pl.ds(0, N)])
  pltpu.sync_copy(slotnode_sh, slotnode_v)
  pltpu.sync_copy(w_sh, w_v)
  pltpu.sync_copy(meta_sh, vec16)
  nslots = vec16[...][0]
  dis0 = dis_v[pl.ds(0, 16)][0]
  pltpu.sync_copy(w1_hbm.at[pl.ds(pl.multiple_of(t * (D_IN * 16), 8),
                                  D_IN * 16)], w1_v)
  pltpu.sync_copy(b1_hbm.at[pl.ds(pl.multiple_of(t * 16, 8), 16)], b1_v)

  def _process16(srcv, slotv, nrmv):
    idxg[...] = srcv
    slotg[...] = slotv
    pltpu.sync_copy(x_hbm.at[idxg], rows_v)

    for l in range(16):
      nl = nrmv[l]

      def _b(b, c2, l=l, nl=nl):
        v = rows_v[l, pl.ds(b * 16, 16)]
        rows_v[l, pl.ds(b * 16, 16)] = v * nl
        return c2
      lax.fori_loop(0, 8, _b, 0)
    pltpu.sync_copy(rows_v, agg_sh.at[slotg], add=True)

  ngroups = (nslots + SMAX - 1) // SMAX

  def _group(g, c):
    glo = g * SMAX
    gcount = jnp.minimum(nslots - glo, jnp.int32(SMAX))

    for l in range(16):
      def _zr2(b, c2, l=l):
        rows_v[l, pl.ds(b * 16, 16)] = fzero16
        return c2
      lax.fori_loop(0, 8, _zr2, 0)

    mv = (gcount + 15) // 16

    def _za(k, c2):
      mchunk = k * 16 + t
      rvec = mchunk * 16 + iota
      rz = jnp.where(rvec < gcount, rvec, jnp.int32(SMAX))
      slotg[...] = rz
      pltpu.sync_copy(rows_v, agg_sh.at[slotg])
      return c2
    lax.fori_loop(0, jnp.maximum(0, (mv - t + 15) // 16), _za, 0)
    plsc.subcore_barrier()

    def _chunk_e(db, sb, cc):
      def _evec(o, mcv, d, f):
        gs = f - 1 - glo
        m = (f > 0) & (gs >= 0) & (gs < gcount)
        s = sb[pl.ds(o, 16)]
        nrm = plsc.load_gather(dis_v, [s]) * plsc.load_gather(dis_v, [d])
        pc = plsc.cumsum(m.astype(jnp.int32))
        pos = pc - 1 + mcv
        plsc.store_scatter(msrc, [pos], s, mask=m)
        plsc.store_scatter(mslot, [pos], gs, mask=m)
        plsc.store_scatter(mnrm, [pos], nrm, mask=m)
        return mcv + plsc.all_reduce_population_count(m)

      def _pair(i, mcv):
        o = i * 32
        d1 = db[pl.ds(o, 16)]
        d2 = db[pl.ds(o + 16, 16)]
        f1 = plsc.load_gather(flag_v, [d1])
        f2 = plsc.load_gather(flag_v, [d2])
        mcv = _evec(o, mcv, d1, f1)
        return _evec(o + 16, mcv, d2, f2)
      mcv = lax.fori_loop(0, VPC // 2, _pair, izero16, unroll=2)
      o = (VPC - 1) * 16
      d = db[pl.ds(o, 16)]
      mcv = _evec(o, mcv, d, plsc.load_gather(flag_v, [d]))
      mcnt = mcv[0]

      flo = (mcnt // 16) * 16
      padm = (iota + flo) >= mcnt
      plsc.store_scatter(msrc, [iota + flo], izero16, mask=padm)
      plsc.store_scatter(mslot, [iota + flo],
                         jnp.full((16,), SMAX, jnp.int32), mask=padm)
      plsc.store_scatter(mnrm, [iota + flo], fzero16, mask=padm)

      def _bat(r, c2):
        _process16(
            msrc[pl.ds(r * 16, 16)],
            mslot[pl.ds(r * 16, 16)],
            mnrm[pl.ds(r * 16, 16)],
        )
        return c2
      lax.fori_loop(0, (mcnt + 15) // 16, _bat, 0)
      return cc

    _scan_pipe(_chunk_e, 0)

    gv = (gcount + 15) // 16

    def _selfk(k, c2):
      v = k * 16 + t
      gslot = v * 16 + iota
      jvec = glo + gslot
      m = gslot < gcount
      nodes = plsc.load_gather(slotnode_v, [jvec], mask=m)
      nodes = jnp.where(m, nodes, 0)
      dv = plsc.load_gather(dis_v, [nodes])
      nrm = jnp.where(m, dv * dv, fzero16)
      slots = jnp.where(m, gslot, jnp.int32(SMAX))
      _process16(nodes, slots, nrm)
      return c2
    lax.fori_loop(0, jnp.maximum(0, (gv - t + 15) // 16), _selfk, 0)
    plsc.subcore_barrier()

    def _fb(r0, c2):
      rvec = r0 * 16 + iota
      rz = jnp.where(rvec < gcount, rvec, 0)
      idxg[...] = rz
      pltpu.sync_copy(agg_sh.at[idxg], rows_v)
      zreg = zblk[...]
      for l in range(16):
        acc = b1_v[...]

        def _kv(kv, acc, l=l):
          av = rows_v[l, pl.ds(kv * 16, 16)]
          for lane in range(16):
            acc = acc + av[lane] * w1_v[pl.ds((kv * 16 + lane) * 16, 16)]
          return acc
        acc = lax.fori_loop(0, D_IN // 16, _kv, acc)
        h = jnp.maximum(acc, 0.0)
        j = glo + r0 * 16 + l
        valid = (r0 * 16 + l < gcount).astype(jnp.float32)
        wj = w_v[pl.ds(j, 16)][0]
        wt = (dis0 * wj
              + jnp.where(j == 0, dis0 * dis0, jnp.float32(0.0))) * valid
        zreg = zreg + wt * h
      zblk[...] = zreg
      return c2
    lax.fori_loop(0, (gcount + 15) // 16, _fb, 0)
    plsc.subcore_barrier()
    return c

  lax.fori_loop(0, ngroups, _group, 0)

  pltpu.sync_copy(zblk, z_sh.at[pl.ds(pl.multiple_of(t * 16, 8), 16)])
  plsc.subcore_barrier()

  @pl.when(t == 0)
  def _finish():
    pltpu.sync_copy(z_sh, zfull)
    pltpu.sync_copy(zfull, z_hbm)


_sc_kernel = pl.kernel(
    _sc_body,
    out_type=(
        jax.ShapeDtypeStruct((D_H,), jnp.float32),
        jax.ShapeDtypeStruct((E,), jnp.int32),
        jax.ShapeDtypeStruct((T * NPAD,), jnp.float32),
    ),
    mesh=_mesh,
    compiler_params=pltpu.CompilerParams(needs_layout_passes=False),
    scratch_types=[
        pltpu.VMEM((MCAP,), jnp.int32),
        pltpu.VMEM((CHUNK,), jnp.int32),
        pltpu.VMEM((MCAP,), jnp.int32),
        pltpu.VMEM((CHUNK,), jnp.int32),
        pltpu.SemaphoreType.DMA,
        pltpu.SemaphoreType.DMA,
        pltpu.VMEM((N + 16,), jnp.float32),
        pltpu.VMEM((N + 16,), jnp.int32),
        pltpu.VMEM((EPT,), jnp.int32),
        pltpu.VMEM((SCAP,), jnp.int32),
        pltpu.VMEM((SCAP,), jnp.float32),
        pltpu.VMEM((MCAP,), jnp.int32),
        pltpu.VMEM((MCAP,), jnp.int32),
        pltpu.VMEM((MCAP,), jnp.float32),
        pltpu.VMEM((16,), jnp.int32),
        pltpu.VMEM((16,), jnp.int32),
        pltpu.VMEM((16, D_IN), jnp.float32),
        pltpu.VMEM((D_IN * 16,), jnp.float32),
        pltpu.VMEM((16,), jnp.float32),
        pltpu.VMEM((16,), jnp.float32),
        pltpu.VMEM((D_H,), jnp.float32),
        pltpu.VMEM((16,), jnp.int32),
        pltpu.VMEM((T * 16,), jnp.int32),
        pltpu.VMEM((640,), jnp.float32),
        pltpu.VMEM((T * 640,), jnp.float32),
        pltpu.SemaphoreType.DMA,
        pltpu.VMEM_SHARED((NPAD,), jnp.float32),
        pltpu.VMEM_SHARED((N,), jnp.int32),
        pltpu.VMEM_SHARED((SCAP,), jnp.int32),
        pltpu.VMEM_SHARED((SCAP,), jnp.float32),
        pltpu.VMEM_SHARED((16,), jnp.int32),
        pltpu.VMEM_SHARED((T * 16,), jnp.int32),
        pltpu.VMEM_SHARED((SMAX + 8, D_IN), jnp.float32),
        pltpu.VMEM_SHARED((D_H,), jnp.float32),
    ],
)


def _head_body(z_ref, w2_ref, b2_ref, wh1_ref, bh1_ref, wh2_ref, bh2_ref,
               o_ref):
  z = z_ref[...]
  h2 = jnp.maximum(
      jnp.dot(z, w2_ref[...], preferred_element_type=jnp.float32)
      + b2_ref[...], 0.0)
  hid = jnp.maximum(
      jnp.dot(h2, wh1_ref[...], preferred_element_type=jnp.float32)
      + bh1_ref[...], 0.0)
  o_ref[...] = (
      jnp.dot(hid, wh2_ref[...], preferred_element_type=jnp.float32)
      + bh2_ref[...])


_head_call = pl.pallas_call(
    _head_body,
    out_shape=jax.ShapeDtypeStruct((1, D_OUT), jnp.float32),
)


def kernel(x, edge_index, W1, b1, W2, b2, Wh1, bh1, Wh2, bh2):
  w1_blocks = W1.reshape(D_IN, 16, 16).transpose(1, 0, 2).reshape(-1)
  z, _, _ = _sc_kernel(edge_index.reshape(-1), x, w1_blocks, b1)
  q = _head_call(
      z.reshape(1, D_H), W2, b2.reshape(1, D_H),
      Wh1, bh1.reshape(1, D_H), Wh2, bh2.reshape(1, D_OUT))
  return q.reshape(D_OUT)
